# baseline (speedup 1.0000x reference)
import functools

import jax
import jax.numpy as jnp
from jax import lax
from jax.experimental import pallas as pl
from jax.experimental.pallas import tpu as pltpu
from jax.experimental.pallas import tpu_sc as plsc

N = 10000
NPAD = 10240
E = 320000
D = 128
NC = 2
NS = 16
NW = NC * NS
CHUNK = 80
EPAD = 327680
EPW = EPAD // NW
NCHUNK = EPW // CHUNK
NB = 2
NROUND = NCHUNK // NB
RPT = NPAD // NS
DCHUNK = 128
DNCHUNK = EPW // DCHUNK
DFIRE = 16

_mesh = plsc.VectorSubcoreMesh(core_axis_name="c", subcore_axis_name="s")


@functools.partial(
    pl.kernel,
    out_type=jax.ShapeDtypeStruct((NC, NPAD), jnp.float32),
    mesh=_mesh,
    scratch_types=[
        pltpu.VMEM((DNCHUNK, DCHUNK), jnp.int32),
        pltpu.VMEM((DCHUNK,), jnp.float32),
        pltpu.VMEM((RPT,), jnp.float32),
        pltpu.VMEM_SHARED((NPAD,), jnp.float32),
        pltpu.SemaphoreType.DMA,
    ],
)
def _deg_kernel(dst2_hbm, out_hbm, didx, ones_v, zeros_v, acc_s, sem):
    c = lax.axis_index("c")
    s = lax.axis_index("s")
    w = c * NS + s

    for k in range(DCHUNK // 16):
        ones_v[pl.ds(k * 16, 16)] = jnp.ones((16,), jnp.float32)

    def zbody(i, carry):
        zeros_v[pl.ds(i * 16, 16)] = jnp.zeros((16,), jnp.float32)
        return carry

    lax.fori_loop(0, RPT // 16, zbody, 0)
    pltpu.sync_copy(zeros_v, acc_s.at[pl.ds(s * RPT, RPT)])
    pltpu.sync_copy(dst2_hbm.at[w], didx)
    plsc.subcore_barrier()

    def rnd(r, carry):
        def fire(i, carry):
            pltpu.async_copy(ones_v, acc_s.at[didx.at[r * DFIRE + i]], sem, add=True)
            return carry

        lax.fori_loop(0, DFIRE, fire, 0)

        def drain(i, carry):
            pltpu.make_async_copy(ones_v, acc_s.at[didx.at[0]], sem).wait()
            return carry

        lax.fori_loop(0, DFIRE, drain, 0)
        return carry

    lax.fori_loop(0, DNCHUNK // DFIRE, rnd, 0)
    plsc.subcore_barrier()
    pltpu.sync_copy(acc_s.at[pl.ds(s * RPT, RPT)], out_hbm.at[c, pl.ds(s * RPT, RPT)])


@functools.partial(
    pl.kernel,
    out_type=jax.ShapeDtypeStruct((NC, NPAD, D), jnp.float32),
    mesh=_mesh,
    scratch_types=(
        [pltpu.VMEM((CHUNK,), jnp.int32)] * 2
        + [pltpu.VMEM((CHUNK, D), jnp.float32)]
        + [pltpu.SemaphoreType.DMA]
        + [pltpu.VMEM_SHARED((NPAD, D), jnp.float32)]
    ),
)
def _segsum_kernel(src_hbm, dst_hbm, y_hbm, out_hbm, *scratch):
    sidx, didx, rows_v, sem, acc_s = scratch

    c = lax.axis_index("c")
    s = lax.axis_index("s")
    w = c * NS + s

    def zbody(i, carry):
        for j in range(D // 16):
            rows_v[i, pl.ds(j * 16, 16)] = jnp.zeros((16,), jnp.float32)
        return carry

    lax.fori_loop(0, CHUNK, zbody, 0)
    for k in range(RPT // CHUNK):
        pltpu.sync_copy(rows_v, acc_s.at[pl.ds(s * RPT + k * CHUNK, CHUNK)])
    plsc.subcore_barrier()

    def body(i, carry):
        base = w * EPW + i * CHUNK
        pltpu.sync_copy(src_hbm.at[pl.ds(base, CHUNK)], sidx)
        pltpu.sync_copy(dst_hbm.at[pl.ds(base, CHUNK)], didx)
        pltpu.async_copy(y_hbm.at[sidx], rows_v, sem).wait()
        pltpu.sync_copy(rows_v, acc_s.at[didx], add=True)
        return carry

    lax.fori_loop(0, NCHUNK, body, 0)
    plsc.subcore_barrier()
    pltpu.sync_copy(acc_s.at[pl.ds(s * RPT, RPT)], out_hbm.at[c, pl.ds(s * RPT, RPT)])


BN = 1000
_PREC = lax.Precision.HIGHEST


def _dis_of(dp_ref):
    deg = dp_ref[0] + dp_ref[1] + 1.0
    return lax.rsqrt(deg)


def _xw_body(dp_ref, x_ref, w_ref, o_ref):
    dis = _dis_of(dp_ref)
    xw = jnp.dot(x_ref[...], w_ref[...], preferred_element_type=jnp.float32,
                 precision=_PREC)
    o_ref[...] = xw * dis


_xw_kernel = pl.pallas_call(
    _xw_body,
    grid=(N // BN,),
    in_specs=[
        pl.BlockSpec((NC, BN, 1), lambda i: (0, i, 0)),
        pl.BlockSpec((BN, D), lambda i: (i, 0)),
        pl.BlockSpec((D, D), lambda i: (0, 0)),
    ],
    out_specs=pl.BlockSpec((BN, D), lambda i: (i, 0)),
    out_shape=jax.ShapeDtypeStruct((N, D), jnp.float32),
)


def _mid_body(dp_ref, z_ref, y_ref, b_ref, w_ref, o_ref):
    dis = _dis_of(dp_ref)
    t = (z_ref[0] + z_ref[1] + y_ref[...]) * dis + b_ref[...]
    h = jnp.maximum(t, 0.0)
    o_ref[...] = jnp.dot(h, w_ref[...], preferred_element_type=jnp.float32,
                         precision=_PREC) * dis


_mid_kernel = pl.pallas_call(
    _mid_body,
    grid=(N // BN,),
    in_specs=[
        pl.BlockSpec((NC, BN, 1), lambda i: (0, i, 0)),
        pl.BlockSpec((NC, BN, D), lambda i: (0, i, 0)),
        pl.BlockSpec((BN, D), lambda i: (i, 0)),
        pl.BlockSpec((1, D), lambda i: (0, 0)),
        pl.BlockSpec((D, D), lambda i: (0, 0)),
    ],
    out_specs=pl.BlockSpec((BN, D), lambda i: (i, 0)),
    out_shape=jax.ShapeDtypeStruct((N, D), jnp.float32),
)


def _fin_body(dp_ref, z_ref, y_ref, b_ref, o_ref):
    dis = _dis_of(dp_ref)
    o_ref[...] = (z_ref[0] + z_ref[1] + y_ref[...]) * dis + b_ref[...]


_fin_kernel = pl.pallas_call(
    _fin_body,
    grid=(N // BN,),
    in_specs=[
        pl.BlockSpec((NC, BN, 1), lambda i: (0, i, 0)),
        pl.BlockSpec((NC, BN, D), lambda i: (0, i, 0)),
        pl.BlockSpec((BN, D), lambda i: (i, 0)),
        pl.BlockSpec((1, D), lambda i: (0, 0)),
    ],
    out_specs=pl.BlockSpec((BN, D), lambda i: (i, 0)),
    out_shape=jax.ShapeDtypeStruct((N, D), jnp.float32),
)


def kernel(x, edge_index, W1, b1, W2, b2):
    ei = edge_index.astype(jnp.int32)
    npadding = EPAD - E
    pad_dst = N + (jnp.arange(npadding, dtype=jnp.int32) % (NPAD - N))
    src = jnp.concatenate([ei[0], jnp.zeros((npadding,), jnp.int32)])
    dst = jnp.concatenate([ei[1], pad_dst])
    b1r = b1.reshape(1, D)
    b2r = b2.reshape(1, D)

    deg_parts = _deg_kernel(dst.reshape(NW, DNCHUNK, DCHUNK))
    dp = deg_parts[:, :N].reshape(NC, N, 1)

    y1 = _xw_kernel(dp, x, W1)
    z1 = _segsum_kernel(src, dst, y1)[:, :N, :]
    y2 = _mid_kernel(dp, z1, y1, b1r, W2)
    z2 = _segsum_kernel(src, dst, y2)[:, :N, :]
    return _fin_kernel(dp, z2, y2, b2r)


# speedup vs baseline: 1.9500x; 1.9500x over previous
import functools

import jax
import jax.numpy as jnp
from jax import lax
from jax.experimental import pallas as pl
from jax.experimental.pallas import tpu as pltpu
from jax.experimental.pallas import tpu_sc as plsc

N = 10000
NPAD = 10240
E = 320000
D = 128
NC = 2
NS = 16
NW = NC * NS
EPW = E // NW
CHUNK = 80
NCHUNK = EPW // CHUNK
RPT = NPAD // NS
DCHUNK = 80
DNCHUNK = EPW // DCHUNK
DFIRE = 25

_mesh = plsc.VectorSubcoreMesh(core_axis_name="c", subcore_axis_name="s")


@functools.partial(
    pl.kernel,
    out_type=jax.ShapeDtypeStruct((NC, NPAD), jnp.float32),
    mesh=_mesh,
    scratch_types=[
        pltpu.VMEM((DNCHUNK, DCHUNK), jnp.int32),
        pltpu.VMEM((DCHUNK,), jnp.float32),
        pltpu.VMEM((RPT,), jnp.float32),
        pltpu.VMEM_SHARED((NPAD,), jnp.float32),
        pltpu.SemaphoreType.DMA,
    ],
)
def _deg_kernel(dst2_hbm, out_hbm, didx, ones_v, zeros_v, acc_s, sem):
    c = lax.axis_index("c")
    s = lax.axis_index("s")
    w = c * NS + s

    for k in range(DCHUNK // 16):
        ones_v[pl.ds(k * 16, 16)] = jnp.ones((16,), jnp.float32)

    def zbody(i, carry):
        zeros_v[pl.ds(i * 16, 16)] = jnp.zeros((16,), jnp.float32)
        return carry

    lax.fori_loop(0, RPT // 16, zbody, 0)
    pltpu.sync_copy(zeros_v, acc_s.at[pl.ds(s * RPT, RPT)])
    pltpu.sync_copy(dst2_hbm.at[w], didx)
    plsc.subcore_barrier()

    def rnd(r, carry):
        def fire(i, carry):
            pltpu.async_copy(ones_v, acc_s.at[didx.at[r * DFIRE + i]], sem, add=True)
            return carry

        lax.fori_loop(0, DFIRE, fire, 0)

        def drain(i, carry):
            pltpu.make_async_copy(ones_v, acc_s.at[didx.at[0]], sem).wait()
            return carry

        lax.fori_loop(0, DFIRE, drain, 0)
        return carry

    lax.fori_loop(0, DNCHUNK // DFIRE, rnd, 0)
    plsc.subcore_barrier()
    pltpu.sync_copy(acc_s.at[pl.ds(s * RPT, RPT)], out_hbm.at[c, pl.ds(s * RPT, RPT)])


@functools.partial(
    pl.kernel,
    out_type=jax.ShapeDtypeStruct((NC, NPAD, D), jnp.float32),
    mesh=_mesh,
    scratch_types=[
        pltpu.VMEM((CHUNK,), jnp.int32),
        pltpu.VMEM((CHUNK,), jnp.int32),
        pltpu.VMEM((CHUNK, D), jnp.float32),
        pltpu.VMEM_SHARED((NPAD, D), jnp.float32),
        pltpu.SemaphoreType.DMA,
    ],
)
def _segsum_kernel(src_hbm, dst_hbm, y_hbm, out_hbm, sidx_v, didx_v, rows_v, acc_s, sem):
    c = lax.axis_index("c")
    s = lax.axis_index("s")
    w = c * NS + s

    def zbody(i, carry):
        for j in range(D // 16):
            rows_v[i, pl.ds(j * 16, 16)] = jnp.zeros((16,), jnp.float32)
        return carry

    lax.fori_loop(0, CHUNK, zbody, 0)
    for k in range(RPT // CHUNK):
        pltpu.sync_copy(rows_v, acc_s.at[pl.ds(s * RPT + k * CHUNK, CHUNK)])
    plsc.subcore_barrier()

    def body(i, carry):
        base = w * EPW + i * CHUNK
        pltpu.sync_copy(src_hbm.at[pl.ds(base, CHUNK)], sidx_v)
        pltpu.sync_copy(dst_hbm.at[pl.ds(base, CHUNK)], didx_v)
        pltpu.async_copy(y_hbm.at[sidx_v], rows_v, sem).wait()
        pltpu.sync_copy(rows_v, acc_s.at[didx_v], add=True)
        return carry

    lax.fori_loop(0, NCHUNK, body, 0)
    plsc.subcore_barrier()
    pltpu.sync_copy(acc_s.at[pl.ds(s * RPT, RPT)], out_hbm.at[c, pl.ds(s * RPT, RPT)])


BN = 1000
_PREC = lax.Precision.HIGHEST


def _dis_of(dp_ref):
    deg = dp_ref[0] + dp_ref[1] + 1.0
    return lax.rsqrt(deg)


def _xw_body(dp_ref, x_ref, w_ref, o_ref):
    dis = _dis_of(dp_ref)
    xw = jnp.dot(x_ref[...], w_ref[...], preferred_element_type=jnp.float32,
                 precision=_PREC)
    o_ref[...] = xw * dis


_xw_kernel = pl.pallas_call(
    _xw_body,
    grid=(N // BN,),
    in_specs=[
        pl.BlockSpec((NC, BN, 1), lambda i: (0, i, 0)),
        pl.BlockSpec((BN, D), lambda i: (i, 0)),
        pl.BlockSpec((D, D), lambda i: (0, 0)),
    ],
    out_specs=pl.BlockSpec((BN, D), lambda i: (i, 0)),
    out_shape=jax.ShapeDtypeStruct((N, D), jnp.float32),
)


def _mid_body(dp_ref, z_ref, y_ref, b_ref, w_ref, o_ref):
    dis = _dis_of(dp_ref)
    t = (z_ref[0] + z_ref[1] + y_ref[...]) * dis + b_ref[...]
    h = jnp.maximum(t, 0.0)
    o_ref[...] = jnp.dot(h, w_ref[...], preferred_element_type=jnp.float32,
                         precision=_PREC) * dis


_mid_kernel = pl.pallas_call(
    _mid_body,
    grid=(N // BN,),
    in_specs=[
        pl.BlockSpec((NC, BN, 1), lambda i: (0, i, 0)),
        pl.BlockSpec((NC, BN, D), lambda i: (0, i, 0)),
        pl.BlockSpec((BN, D), lambda i: (i, 0)),
        pl.BlockSpec((1, D), lambda i: (0, 0)),
        pl.BlockSpec((D, D), lambda i: (0, 0)),
    ],
    out_specs=pl.BlockSpec((BN, D), lambda i: (i, 0)),
    out_shape=jax.ShapeDtypeStruct((N, D), jnp.float32),
)


def _fin_body(dp_ref, z_ref, y_ref, b_ref, o_ref):
    dis = _dis_of(dp_ref)
    o_ref[...] = (z_ref[0] + z_ref[1] + y_ref[...]) * dis + b_ref[...]


_fin_kernel = pl.pallas_call(
    _fin_body,
    grid=(N // BN,),
    in_specs=[
        pl.BlockSpec((NC, BN, 1), lambda i: (0, i, 0)),
        pl.BlockSpec((NC, BN, D), lambda i: (0, i, 0)),
        pl.BlockSpec((BN, D), lambda i: (i, 0)),
        pl.BlockSpec((1, D), lambda i: (0, 0)),
    ],
    out_specs=pl.BlockSpec((BN, D), lambda i: (i, 0)),
    out_shape=jax.ShapeDtypeStruct((N, D), jnp.float32),
)


def kernel(x, edge_index, W1, b1, W2, b2):
    ei = edge_index.astype(jnp.int32)
    src = ei[0]
    dst = ei[1]
    b1r = b1.reshape(1, D)
    b2r = b2.reshape(1, D)

    deg_parts = _deg_kernel(dst.reshape(NW, DNCHUNK, DCHUNK))
    dp = deg_parts[:, :N].reshape(NC, N, 1)

    y1 = _xw_kernel(dp, x, W1)
    z1 = _segsum_kernel(src, dst, y1)[:, :N, :]
    y2 = _mid_kernel(dp, z1, y1, b1r, W2)
    z2 = _segsum_kernel(src, dst, y2)[:, :N, :]
    return _fin_kernel(dp, z2, y2, b2r)
